# baseline (speedup 1.0000x reference)
import functools

import jax
import jax.numpy as jnp
from jax import lax
from jax.experimental import pallas as pl
from jax.experimental.pallas import tpu as pltpu
from jax.experimental.pallas import tpu_sc as plsc

EMBED_D = 64
BATCH = 4096
SEQ = 200
NUM_WORKERS = 32
ROWS_PER_W = BATCH // NUM_WORKERS
NB = 8
NGROUP = ROWS_PER_W // NB

_mesh = plsc.VectorSubcoreMesh(core_axis_name="c", subcore_axis_name="s")


@functools.partial(
    pl.kernel,
    mesh=_mesh,
    out_type=jax.ShapeDtypeStruct((BATCH, SEQ, EMBED_D), jnp.float32),
    scratch_types=[
        pltpu.VMEM((ROWS_PER_W, SEQ), jnp.int32),
        [pltpu.VMEM((SEQ, EMBED_D), jnp.float32) for _ in range(NB)],
        [pltpu.SemaphoreType.DMA for _ in range(NB)],
        [pltpu.SemaphoreType.DMA for _ in range(NB)],
    ],
    compiler_params=pltpu.CompilerParams(use_tc_tiling_on_sc=False),
)
def _gather_kernel(table_hbm, x_hbm, out_hbm, idx_v, rows, gsem, ssem):
    wid = lax.axis_index("s") * 2 + lax.axis_index("c")
    base = wid * ROWS_PER_W
    pltpu.sync_copy(x_hbm.at[pl.ds(base, ROWS_PER_W)], idx_v)

    def gather(r, b):
        return pltpu.make_async_copy(table_hbm.at[idx_v.at[r]], rows[b], gsem[b])

    def store(r, b):
        return pltpu.make_async_copy(rows[b], out_hbm.at[base + r], ssem[b])

    def body(p, _):
        r0 = p * NB
        for b in range(NB):
            @pl.when(p > 0)
            def _():
                store(r0 + b - NB, b).wait()
            gather(r0 + b, b).start()
        for b in range(NB):
            gather(r0 + b, b).wait()
            store(r0 + b, b).start()
        return ()

    lax.fori_loop(0, NGROUP, body, ())
    for b in range(NB):
        store(ROWS_PER_W - NB + b, b).wait()


def kernel(x, embeddings):
    return _gather_kernel(embeddings, x)
